# Initial kernel scaffold; baseline (speedup 1.0000x reference)
import jax
import jax.numpy as jnp
from jax.experimental import pallas as pl


def kernel(x, w1, b1, w2, b2, gamma, beta):
    raise NotImplementedError("write your pallas kernel here")


# speedup vs baseline: 1.0613x; 1.0613x over previous
import functools

import jax
import jax.numpy as jnp
from jax.experimental import pallas as pl
from jax.experimental.pallas import tpu as pltpu


def _afn_block_kernel(x_ref, w1_ref, b1_ref, w2_ref, b2_ref, g_ref, bt_ref,
                      o_ref, *, eps, hw, ch):
    inv_hw = 1.0 / hw
    inv_nm1 = 1.0 / max(hw - 1, 1)

    xf = x_ref[...].astype(jnp.float32)
    s = jnp.sum(xf, axis=2)
    ss = jnp.sum(xf * xf, axis=2)

    mean = s * inv_hw
    var = jnp.maximum((ss - mean * s) * inv_nm1, 0.0)
    inv_std = jax.lax.rsqrt(var + eps)

    dn = (((1,), (1,)), ((), ()))
    h1 = jnp.maximum(
        jax.lax.dot_general(mean, w1_ref[...], dn,
                            preferred_element_type=jnp.float32) + b1_ref[...],
        0.0)
    a = jax.lax.dot_general(h1, w2_ref[...], dn,
                            preferred_element_type=jnp.float32) + b2_ref[...]
    a_g = a[:, :ch]
    a_b = a[:, ch:]

    scale = (1.0 + a_g) * g_ref[...] * inv_std
    off = a_b * bt_ref[...] - scale * mean

    o_ref[...] = (x_ref[...].astype(jnp.float32) * scale[:, :, None]
                  + off[:, :, None]).astype(o_ref.dtype)


def kernel(x, w1, b1, w2, b2, gamma, beta, *, eps=1e-5):
    B, C, H, W = x.shape
    Cq = C // 4
    HW = H * W

    xr = x.reshape(B, C, HW)
    f32 = jnp.float32
    w1m = w1.reshape(Cq, C).astype(f32)
    b1r = b1.reshape(1, Cq).astype(f32)
    w2m = w2.reshape(2 * C, Cq).astype(f32)
    b2r = b2.reshape(1, 2 * C).astype(f32)
    grow = gamma.reshape(1, C).astype(f32)
    brow = beta.reshape(1, C).astype(f32)

    itemsize = x.dtype.itemsize
    per_b = C * HW * itemsize
    budget = 40 << 20
    bb = 1
    for cand in range(1, B + 1):
        if B % cand:
            continue
        if 2 * cand * per_b + 2 * cand * C * HW * 4 > budget:
            break
        if B // cand < 4:
            break
        bb = cand
    nb = B // bb

    vmem_limit = int(min(60 << 20, 2 * bb * per_b + 2 * bb * C * HW * 4
                         + (8 << 20)))

    const = lambda i: (0, 0)
    out = pl.pallas_call(
        functools.partial(_afn_block_kernel, eps=eps, hw=HW, ch=C),
        out_shape=jax.ShapeDtypeStruct((B, C, HW), x.dtype),
        grid=(nb,),
        in_specs=[
            pl.BlockSpec((bb, C, HW), lambda i: (i, 0, 0)),
            pl.BlockSpec((Cq, C), const),
            pl.BlockSpec((1, Cq), const),
            pl.BlockSpec((2 * C, Cq), const),
            pl.BlockSpec((1, 2 * C), const),
            pl.BlockSpec((1, C), const),
            pl.BlockSpec((1, C), const),
        ],
        out_specs=pl.BlockSpec((bb, C, HW), lambda i: (i, 0, 0)),
        compiler_params=pltpu.CompilerParams(
            dimension_semantics=("parallel",),
            vmem_limit_bytes=vmem_limit),
    )(xr, w1m, b1r, w2m, b2r, grow, brow)

    return out.reshape(B, C, H, W)
